# baseline (speedup 1.0000x reference)
import functools

import jax
import jax.numpy as jnp
from jax import lax
from jax.experimental import pallas as pl
from jax.experimental.pallas import tpu as pltpu
from jax.experimental.pallas import tpu_sc as plsc

V0, V1, V2 = 12, 15, 7
D = 32
NROWS = V0 * V1 * V2
NPAD = 1280
NC, NS = 2, 16
NW = NC * NS


def _fuse_body(w0_ref, w1_ref, w2_ref, out_ref):
    r0 = lax.broadcasted_iota(jnp.int32, (NPAD, V0), 0)
    k0 = lax.broadcasted_iota(jnp.int32, (NPAD, V0), 1)
    oh0 = (r0 % V0 == k0).astype(jnp.float32)
    r1 = lax.broadcasted_iota(jnp.int32, (NPAD, V1), 0)
    k1 = lax.broadcasted_iota(jnp.int32, (NPAD, V1), 1)
    oh1 = ((r1 // V0) % V1 == k1).astype(jnp.float32)
    r2 = lax.broadcasted_iota(jnp.int32, (NPAD, V2), 0)
    k2 = lax.broadcasted_iota(jnp.int32, (NPAD, V2), 1)
    oh2 = (r2 // (V0 * V1) == k2).astype(jnp.float32)
    out_ref[...] = (
        jnp.dot(oh0, w0_ref[...], preferred_element_type=jnp.float32,
                precision=lax.Precision.HIGHEST)
        + jnp.dot(oh1, w1_ref[...], preferred_element_type=jnp.float32,
                  precision=lax.Precision.HIGHEST)
        + jnp.dot(oh2, w2_ref[...], preferred_element_type=jnp.float32,
                  precision=lax.Precision.HIGHEST)
    )


def _build_fused(W0, W1, W2):
    return pl.pallas_call(
        _fuse_body,
        out_shape=jax.ShapeDtypeStruct((NPAD, D), jnp.float32),
    )(W0, W1, W2)


@functools.partial(jax.jit, static_argnames=("E", "S"))
def _sc_gather(wf_flat, c32, E, S):
    TL = S // 128
    NCH = E // S

    mesh = plsc.VectorSubcoreMesh(
        core_axis_name="c", subcore_axis_name="s",
        num_cores=NC, num_subcores=NS)

    OSZ = 4 * TL * 1024

    def body(wf_hbm, c_hbm, out_hbm, wf_v, c_v, out_v, stg,
             sem_c, sem_out):
        wid = lax.axis_index("s") * NC + lax.axis_index("c")
        lane = lax.iota(jnp.int32, 16)
        idxv = lane * 33
        spl = [jnp.full((16,), k, jnp.int32) for k in range(16)]
        pltpu.sync_copy(wf_hbm, wf_v)

        cnt = (NCH - wid + NW - 1) // NW

        def c_copy(t):
            tt = jnp.minimum(t, cnt - 1)
            ch = wid + tt * NW
            return pltpu.async_copy(
                c_hbm.at[pl.ds(ch * S, S)],
                c_v.at[pl.ds((t % 2) * S, S)],
                sem_c)

        c_copy(0).wait()

        def chunk(t, _):
            p = t % 2
            ch = wid + t * NW
            c_copy(t + 1)

            obase = p * OSZ
            cbase = p * S

            @plsc.parallel_loop(0, TL * 8, unroll=4)
            def _(i):
                cvq = c_v[pl.ds(cbase + i * 16, 16)]
                for k in range(16):
                    b = cvq.at[spl[k]].get(
                        mode="promise_in_bounds") + lane
                    stg[pl.ds(i * 528 + k * 33, 16)] = \
                        plsc.load_gather(wf_v, [b])
                    stg[pl.ds(i * 528 + k * 33 + 16, 16)] = \
                        plsc.load_gather(wf_v, [b + 16])

            @plsc.parallel_loop(0, TL * D, unroll=8)
            def _(j):
                tl = j // D
                d = j % D
                o = obase + (d // 8) * (TL * 1024) + tl * 1024 \
                    + (d % 8) * 128
                sb = tl * 4224 + d
                for q in range(8):
                    v = plsc.load_gather(stg, [idxv + (sb + q * 528)])
                    out_v[pl.ds(o + q * 16, 16)] = v

            copies = [
                pltpu.async_copy(
                    out_v.at[pl.ds(obase + dg * (TL * 1024), TL * 1024)],
                    out_hbm.at[pl.ds(dg * (8 * E) + ch * (TL * 1024),
                                     TL * 1024)],
                    sem_out)
                for dg in range(4)
            ]

            @pl.when(t > 0)
            def _():
                for _ in range(4):
                    pltpu.make_async_copy(
                        out_v.at[pl.ds(0, TL * 1024)],
                        out_hbm.at[pl.ds(0, TL * 1024)],
                        sem_out).wait()

            pltpu.make_async_copy(
                c_hbm.at[pl.ds(0, S)], c_v.at[pl.ds(0, S)], sem_c).wait()
            return 0

        lax.fori_loop(0, cnt, chunk, 0, unroll=False)
        for _ in range(4):
            pltpu.make_async_copy(
                out_v.at[pl.ds(0, TL * 1024)],
                out_hbm.at[pl.ds(0, TL * 1024)],
                sem_out).wait()

    return pl.kernel(
        body,
        out_type=jax.ShapeDtypeStruct((D * E,), jnp.float32),
        mesh=mesh,
        compiler_params=pltpu.CompilerParams(needs_layout_passes=False),
        scratch_types=[
            pltpu.VMEM((NPAD * D,), jnp.float32),
            pltpu.VMEM((2 * S,), jnp.int32),
            pltpu.VMEM((2 * 4 * TL * 1024,), jnp.float32),
            pltpu.VMEM((TL * 128 * 33,), jnp.float32),
            pltpu.SemaphoreType.DMA,
            pltpu.SemaphoreType.DMA,
        ],
    )(wf_flat, c32)


def kernel(edge_features, W0, W1, W2):
    E = edge_features.shape[0]
    S = 640
    assert E % S == 0
    ef = edge_features.astype(jnp.int32)
    c32 = (ef[:, 0] + ef[:, 1] * V0 + ef[:, 2] * (V0 * V1)) * D
    wf = _build_fused(W0, W1, W2).reshape(-1)
    out = _sc_gather(wf, c32, E, S)
    return out.reshape(4, E // 128, 8, 128).transpose(1, 3, 0, 2).reshape(E, D)


# speedup vs baseline: 1.3613x; 1.3613x over previous
import functools

import jax
import jax.numpy as jnp
from jax import lax
from jax.experimental import pallas as pl
from jax.experimental.pallas import tpu as pltpu
from jax.experimental.pallas import tpu_sc as plsc

V0, V1, V2 = 12, 15, 7
D = 32
NROWS = V0 * V1 * V2
NPAD = 1280
NC, NS = 2, 16
NW = NC * NS


def _fuse_body(w0_ref, w1_ref, w2_ref, out_ref):
    r0 = lax.broadcasted_iota(jnp.int32, (NPAD, V0), 0)
    k0 = lax.broadcasted_iota(jnp.int32, (NPAD, V0), 1)
    oh0 = (r0 % V0 == k0).astype(jnp.float32)
    r1 = lax.broadcasted_iota(jnp.int32, (NPAD, V1), 0)
    k1 = lax.broadcasted_iota(jnp.int32, (NPAD, V1), 1)
    oh1 = ((r1 // V0) % V1 == k1).astype(jnp.float32)
    r2 = lax.broadcasted_iota(jnp.int32, (NPAD, V2), 0)
    k2 = lax.broadcasted_iota(jnp.int32, (NPAD, V2), 1)
    oh2 = (r2 // (V0 * V1) == k2).astype(jnp.float32)
    out_ref[...] = (
        jnp.dot(oh0, w0_ref[...], preferred_element_type=jnp.float32,
                precision=lax.Precision.HIGHEST)
        + jnp.dot(oh1, w1_ref[...], preferred_element_type=jnp.float32,
                  precision=lax.Precision.HIGHEST)
        + jnp.dot(oh2, w2_ref[...], preferred_element_type=jnp.float32,
                  precision=lax.Precision.HIGHEST)
    )


def _build_fused(W0, W1, W2):
    return pl.pallas_call(
        _fuse_body,
        out_shape=jax.ShapeDtypeStruct((NPAD, D), jnp.float32),
    )(W0, W1, W2)


@functools.partial(jax.jit, static_argnames=("E", "S"))
def _sc_gather(wf_flat, c32, E, S):
    TL = S // 128
    NCH = E // S

    mesh = plsc.VectorSubcoreMesh(
        core_axis_name="c", subcore_axis_name="s",
        num_cores=NC, num_subcores=NS)

    OSZ = 4 * TL * 1024

    def body(wf_hbm, c_hbm, out_hbm, wf_v, c_v, out_v, stg,
             sem_c, sem_out):
        wid = lax.axis_index("s") * NC + lax.axis_index("c")
        lane = lax.iota(jnp.int32, 16)
        idxv = lane * 33
        spl = [jnp.full((16,), k, jnp.int32) for k in range(16)]
        pltpu.sync_copy(wf_hbm, wf_v)

        cnt = (NCH - wid + NW - 1) // NW

        def c_copy(t):
            tt = jnp.minimum(t, cnt - 1)
            ch = wid + tt * NW
            return pltpu.async_copy(
                c_hbm.at[pl.ds(ch * S, S)],
                c_v.at[pl.ds((t % 2) * S, S)],
                sem_c)

        c_copy(0).wait()

        def chunk(t, _):
            p = t % 2
            ch = wid + t * NW
            c_copy(t + 1)

            obase = p * OSZ
            cbase = p * S

            def tlbody(tl, _):
                @plsc.parallel_loop(0, 8, unroll=4)
                def _(q):
                    cvq = c_v[pl.ds(cbase + tl * 128 + q * 16, 16)]
                    for k in range(16):
                        b = cvq.at[spl[k]].get(
                            mode="promise_in_bounds") + lane
                        stg[pl.ds(q * 528 + k * 33, 16)] = \
                            plsc.load_gather(wf_v, [b])
                        stg[pl.ds(q * 528 + k * 33 + 16, 16)] = \
                            plsc.load_gather(wf_v, [b + 16])

                @plsc.parallel_loop(0, D, unroll=8)
                def _(d):
                    o = obase + (d // 8) * (TL * 1024) + tl * 1024 \
                        + (d % 8) * 128
                    for q in range(8):
                        v = plsc.load_gather(stg, [idxv + (q * 528 + d)])
                        out_v[pl.ds(o + q * 16, 16)] = v
                return 0

            lax.fori_loop(0, TL, tlbody, 0, unroll=False)

            copies = [
                pltpu.async_copy(
                    out_v.at[pl.ds(obase + dg * (TL * 1024), TL * 1024)],
                    out_hbm.at[pl.ds(dg * (8 * E) + ch * (TL * 1024),
                                     TL * 1024)],
                    sem_out)
                for dg in range(4)
            ]

            @pl.when(t > 0)
            def _():
                for _ in range(4):
                    pltpu.make_async_copy(
                        out_v.at[pl.ds(0, TL * 1024)],
                        out_hbm.at[pl.ds(0, TL * 1024)],
                        sem_out).wait()

            pltpu.make_async_copy(
                c_hbm.at[pl.ds(0, S)], c_v.at[pl.ds(0, S)], sem_c).wait()
            return 0

        lax.fori_loop(0, cnt, chunk, 0, unroll=False)
        for _ in range(4):
            pltpu.make_async_copy(
                out_v.at[pl.ds(0, TL * 1024)],
                out_hbm.at[pl.ds(0, TL * 1024)],
                sem_out).wait()

    return pl.kernel(
        body,
        out_type=jax.ShapeDtypeStruct((D * E,), jnp.float32),
        mesh=mesh,
        compiler_params=pltpu.CompilerParams(needs_layout_passes=False),
        scratch_types=[
            pltpu.VMEM((NPAD * D,), jnp.float32),
            pltpu.VMEM((2 * S,), jnp.int32),
            pltpu.VMEM((2 * 4 * TL * 1024,), jnp.float32),
            pltpu.VMEM((128 * 33,), jnp.float32),
            pltpu.SemaphoreType.DMA,
            pltpu.SemaphoreType.DMA,
        ],
    )(wf_flat, c32)


def kernel(edge_features, W0, W1, W2):
    E = edge_features.shape[0]
    S = 640
    assert E % S == 0
    ef = edge_features.astype(jnp.int32)
    c32 = (ef[:, 0] + ef[:, 1] * V0 + ef[:, 2] * (V0 * V1)) * D
    wf = _build_fused(W0, W1, W2).reshape(-1)
    out = _sc_gather(wf, c32, E, S)
    return out.reshape(4, E // 128, 8, 128).transpose(1, 3, 0, 2).reshape(E, D)
